# Initial kernel scaffold; baseline (speedup 1.0000x reference)
import jax
import jax.numpy as jnp
from jax.experimental import pallas as pl


def kernel(measurements, tMP, tKF, idxMP, idxKF):
    raise NotImplementedError("write your pallas kernel here")


# speedup vs baseline: 17.8009x; 17.8009x over previous
import functools

import jax
import jax.numpy as jnp
from jax import lax
from jax.experimental import pallas as pl
from jax.experimental.pallas import tpu as pltpu
from jax.experimental.pallas import tpu_sc as plsc

_N_KF = 8
_N_MP = 32
_FX = 320.0
_FY = 320.0
_CX = 320.0
_CY = 240.0
_L = 16


@functools.lru_cache(maxsize=None)
def _build_sc_call(m2: int):
    info = plsc.get_sparse_core_info()
    nc, ns = info.num_cores, info.num_subcores
    nw = nc * ns
    assert m2 % (nw * _L) == 0
    ch = m2 // nw
    n_it = ch // _L

    mesh = plsc.VectorSubcoreMesh(core_axis_name="c", subcore_axis_name="s")

    @functools.partial(
        pl.kernel,
        out_type=jax.ShapeDtypeStruct((m2,), jnp.float32),
        mesh=mesh,
        scratch_types=[
            pltpu.VMEM((ch,), jnp.int32),
            pltpu.VMEM((ch,), jnp.float32),
            pltpu.VMEM((2 * _N_KF * _N_MP,), jnp.float32),
            pltpu.VMEM((_L,), jnp.int32),
            pltpu.VMEM((_N_MP,), jnp.int32),
            pltpu.VMEM((3 * _N_MP,), jnp.float32),
            pltpu.VMEM((_N_KF * 16,), jnp.float32),
            pltpu.VMEM((_L,), jnp.int32),
            pltpu.VMEM((_N_MP,), jnp.int32),
        ],
        compiler_params=pltpu.CompilerParams(needs_layout_passes=False),
    )
    def sc_call(meas_hbm, tmpt_hbm, tkf_hbm, idxkf_hbm, idxmp_hbm, out_hbm,
                inb, outb, tbl, poskf, posmp, tmpt, tkf, idxkf, idxmp):
        wid = lax.axis_index("s") * nc + lax.axis_index("c")
        base = wid * jnp.int32(ch)

        pltpu.sync_copy(meas_hbm.at[pl.ds(base, ch)], inb)
        pltpu.sync_copy(tmpt_hbm, tmpt)
        pltpu.sync_copy(tkf_hbm, tkf)
        pltpu.sync_copy(idxkf_hbm, idxkf)
        pltpu.sync_copy(idxmp_hbm, idxmp)

        iota = lax.iota(jnp.int32, _L)

        plsc.store_scatter(poskf, [idxkf[...]], iota)
        plsc.store_scatter(posmp, [idxmp[pl.ds(0, _L)]], iota)
        plsc.store_scatter(posmp, [idxmp[pl.ds(_L, _L)]], iota + _L)

        xp, yp, zp = [], [], []
        for h in range(2):
            pos = posmp[pl.ds(h * _L, _L)]
            xp.append(plsc.load_gather(tmpt, [pos]))
            yp.append(plsc.load_gather(tmpt, [pos + _N_MP]))
            zp.append(plsc.load_gather(tmpt, [pos + 2 * _N_MP]))

        pk = poskf[...]
        for v in range(_N_KF):
            row = tkf[pl.ds(pk[v] * jnp.int32(16), 16)]
            a00 = row[0]; a01 = row[1]; a02 = row[2]; a03 = row[3]
            a10 = row[4]; a11 = row[5]; a12 = row[6]; a13 = row[7]
            a20 = row[8]; a21 = row[9]; a22 = row[10]; a23 = row[11]
            for h in range(2):
                r0 = a00 * xp[h] + a01 * yp[h] + a02 * zp[h] + a03
                r1 = a10 * xp[h] + a11 * yp[h] + a12 * zp[h] + a13
                r2 = a20 * xp[h] + a21 * yp[h] + a22 * zp[h] + a23
                inv = 1.0 / r2
                px = r0 * inv * _FX + _CX
                py = r1 * inv * _FY + _CY
                tb = 2 * (v * _N_MP + h * _L)
                plsc.store_scatter(tbl, [2 * iota + tb], px)
                plsc.store_scatter(tbl, [2 * iota + (tb + 1)], py)

        e_idx = iota & -2
        o_idx = iota | 1
        par = iota & 1

        def body(i, carry):
            off = i * jnp.int32(_L)
            v = inb[pl.ds(off, _L)]
            kf = v.at[e_idx].get(mode="promise_in_bounds")
            mp = v.at[o_idx].get(mode="promise_in_bounds")
            tix = jnp.left_shift(kf, 6) + jnp.left_shift(mp, 1) + par
            outb[pl.ds(off, _L)] = plsc.load_gather(tbl, [tix])
            return carry

        lax.fori_loop(jnp.int32(0), jnp.int32(n_it), body, jnp.int32(0))

        pltpu.sync_copy(outb, out_hbm.at[pl.ds(base, ch)])

    return sc_call


def kernel(measurements, tMP, tKF, idxMP, idxKF):
    m = measurements.shape[0]
    out_dtype = jnp.promote_types(tMP.dtype, tKF.dtype)
    meas = measurements.astype(jnp.int32).reshape(-1)
    tmpt = tMP.astype(jnp.float32).T.reshape(-1)
    tkf = tKF.astype(jnp.float32).reshape(-1)
    idxkf = jnp.concatenate(
        [idxKF.astype(jnp.int32),
         jnp.arange(_N_KF, _L, dtype=jnp.int32)])
    idxmp = idxMP.astype(jnp.int32)
    out = _build_sc_call(meas.shape[0])(meas, tmpt, tkf, idxkf, idxmp)
    return out.reshape(m, 2).astype(out_dtype)


# speedup vs baseline: 1046.9862x; 58.8163x over previous
import functools

import jax
import jax.numpy as jnp
from jax import lax
from jax.experimental import pallas as pl
from jax.experimental.pallas import tpu as pltpu
from jax.experimental.pallas import tpu_sc as plsc

_N_KF = 8
_N_MP = 32
_FX = 320.0
_FY = 320.0
_CX = 320.0
_CY = 240.0
_L = 16


@functools.lru_cache(maxsize=None)
def _build_sc_call(m: int):
    info = plsc.get_sparse_core_info()
    nc, ns = info.num_cores, info.num_subcores
    nw = nc * ns
    assert m % (nw * _L) == 0
    ch = m // nw
    n_it = ch // _L

    mesh = plsc.VectorSubcoreMesh(core_axis_name="c", subcore_axis_name="s")

    @functools.partial(
        pl.kernel,
        out_type=jax.ShapeDtypeStruct((2 * m,), jnp.float32),
        mesh=mesh,
        scratch_types=[
            pltpu.VMEM((ch,), jnp.int32),
            pltpu.VMEM((ch,), jnp.int32),
            pltpu.VMEM((ch,), jnp.float32),
            pltpu.VMEM((ch,), jnp.float32),
            pltpu.VMEM((_N_KF * _N_MP,), jnp.float32),
            pltpu.VMEM((_N_KF * _N_MP,), jnp.float32),
            pltpu.VMEM((_L,), jnp.int32),
            pltpu.VMEM((_N_MP,), jnp.int32),
            pltpu.VMEM((3 * _N_MP,), jnp.float32),
            pltpu.VMEM((_N_KF * 16,), jnp.float32),
            pltpu.VMEM((_L,), jnp.int32),
            pltpu.VMEM((_N_MP,), jnp.int32),
        ],
        compiler_params=pltpu.CompilerParams(needs_layout_passes=False),
    )
    def sc_call(kf_hbm, mp_hbm, tmpt_hbm, tkf_hbm, idxkf_hbm, idxmp_hbm,
                out_hbm, inkf, inmp, outx, outy, tblx, tbly,
                poskf, posmp, tmpt, tkf, idxkf, idxmp):
        wid = lax.axis_index("s") * nc + lax.axis_index("c")
        base = wid * jnp.int32(ch)

        pltpu.sync_copy(kf_hbm.at[pl.ds(base, ch)], inkf)
        pltpu.sync_copy(mp_hbm.at[pl.ds(base, ch)], inmp)
        pltpu.sync_copy(tmpt_hbm, tmpt)
        pltpu.sync_copy(tkf_hbm, tkf)
        pltpu.sync_copy(idxkf_hbm, idxkf)
        pltpu.sync_copy(idxmp_hbm, idxmp)

        iota = lax.iota(jnp.int32, _L)

        plsc.store_scatter(poskf, [idxkf[...]], iota)
        plsc.store_scatter(posmp, [idxmp[pl.ds(0, _L)]], iota)
        plsc.store_scatter(posmp, [idxmp[pl.ds(_L, _L)]], iota + _L)

        xp, yp, zp = [], [], []
        for h in range(2):
            pos = posmp[pl.ds(h * _L, _L)]
            xp.append(plsc.load_gather(tmpt, [pos]))
            yp.append(plsc.load_gather(tmpt, [pos + _N_MP]))
            zp.append(plsc.load_gather(tmpt, [pos + 2 * _N_MP]))

        pk = poskf[...]
        for v in range(_N_KF):
            row = tkf[pl.ds(pk[v] * jnp.int32(16), 16)]
            a00 = row[0]; a01 = row[1]; a02 = row[2]; a03 = row[3]
            a10 = row[4]; a11 = row[5]; a12 = row[6]; a13 = row[7]
            a20 = row[8]; a21 = row[9]; a22 = row[10]; a23 = row[11]
            for h in range(2):
                r0 = a00 * xp[h] + a01 * yp[h] + a02 * zp[h] + a03
                r1 = a10 * xp[h] + a11 * yp[h] + a12 * zp[h] + a13
                r2 = a20 * xp[h] + a21 * yp[h] + a22 * zp[h] + a23
                inv = 1.0 / r2
                tb = v * _N_MP + h * _L
                tblx[pl.ds(tb, _L)] = r0 * inv * _FX + _CX
                tbly[pl.ds(tb, _L)] = r1 * inv * _FY + _CY

        def body(i, carry):
            off = i * jnp.int32(_L)
            idx = jnp.left_shift(inkf[pl.ds(off, _L)], 5) + inmp[pl.ds(off, _L)]
            outx[pl.ds(off, _L)] = plsc.load_gather(tblx, [idx])
            outy[pl.ds(off, _L)] = plsc.load_gather(tbly, [idx])
            return carry

        lax.fori_loop(jnp.int32(0), jnp.int32(n_it), body, jnp.int32(0))

        pltpu.sync_copy(outx, out_hbm.at[pl.ds(base, ch)])
        pltpu.sync_copy(outy, out_hbm.at[pl.ds(jnp.int32(m) + base, ch)])

    return sc_call


def kernel(measurements, tMP, tKF, idxMP, idxKF):
    m = measurements.shape[0]
    out_dtype = jnp.promote_types(tMP.dtype, tKF.dtype)
    kf = measurements[:, 0].astype(jnp.int32)
    mp = measurements[:, 1].astype(jnp.int32)
    tmpt = tMP.astype(jnp.float32).T.reshape(-1)
    tkf = tKF.astype(jnp.float32).reshape(-1)
    idxkf = jnp.concatenate(
        [idxKF.astype(jnp.int32),
         jnp.arange(_N_KF, _L, dtype=jnp.int32)])
    idxmp = idxMP.astype(jnp.int32)
    out = _build_sc_call(m)(kf, mp, tmpt, tkf, idxkf, idxmp)
    return out.reshape(2, m).transpose(1, 0).astype(out_dtype)
